# baseline (speedup 1.0000x reference)
import jax
import jax.numpy as jnp
from jax import lax
from jax.experimental import pallas as pl
from jax.experimental.pallas import tpu as pltpu
from jax.experimental.pallas import tpu_sc as plsc

_VOCAB = 1000000
_D = 64
_H = 128
_B = 4096
_L = 200

_NC = 2
_NS = 16
_NW = _NC * _NS
_RPT = _B // _NW
_C0 = 128
_C1 = _L - _C0


def _sc_body(idx_hbm, tab_hbm, out_hbm, idx_v, a0, a1, b0, b1, out_v,
             s_a0, s_a1, s_b0, s_b1):
    c = lax.axis_index("c")
    s = lax.axis_index("s")
    wid = s * _NC + c
    rbase = wid * _RPT

    pltpu.sync_copy(idx_hbm.at[pl.ds(rbase, _RPT)], idx_v)

    def gather(row, col, width, buf, sem):
        pltpu.async_copy(tab_hbm.at[idx_v.at[row, pl.ds(col, width)]], buf, sem)

    def wait(width, buf, sem):
        pltpu.make_async_copy(
            tab_hbm.at[idx_v.at[0, pl.ds(0, width)]], buf, sem).wait()

    zero = jnp.zeros((16,), jnp.float32)
    acc_init = (zero,) * 8

    def reduce_rows(buf, n, accs):
        def rb(i, a):
            r0 = 2 * i
            r1 = r0 + 1
            return (
                a[0] + buf[r0, pl.ds(0, 16)],
                a[1] + buf[r0, pl.ds(16, 16)],
                a[2] + buf[r0, pl.ds(32, 16)],
                a[3] + buf[r0, pl.ds(48, 16)],
                a[4] + buf[r1, pl.ds(0, 16)],
                a[5] + buf[r1, pl.ds(16, 16)],
                a[6] + buf[r1, pl.ds(32, 16)],
                a[7] + buf[r1, pl.ds(48, 16)],
            )
        return lax.fori_loop(0, n // 2, rb, accs)

    def store_row(row, accs):
        out_v[row, pl.ds(0, 16)] = accs[0] + accs[4]
        out_v[row, pl.ds(16, 16)] = accs[1] + accs[5]
        out_v[row, pl.ds(32, 16)] = accs[2] + accs[6]
        out_v[row, pl.ds(48, 16)] = accs[3] + accs[7]

    gather(0, 0, _C0, a0, s_a0)
    gather(0, _C0, _C1, a1, s_a1)
    gather(1, 0, _C0, b0, s_b0)
    gather(1, _C0, _C1, b1, s_b1)

    nbb = _RPT // 2

    def iter_body(bb, carry):
        r0 = 2 * bb
        r1 = r0 + 1
        more = bb < nbb - 1

        wait(_C0, a0, s_a0)
        acc = reduce_rows(a0, _C0, acc_init)

        @pl.when(more)
        def _():
            gather(r0 + 2, 0, _C0, a0, s_a0)

        wait(_C1, a1, s_a1)
        acc = reduce_rows(a1, _C1, acc)

        @pl.when(more)
        def _():
            gather(r0 + 2, _C0, _C1, a1, s_a1)

        store_row(r0, acc)

        wait(_C0, b0, s_b0)
        acc = reduce_rows(b0, _C0, acc_init)

        @pl.when(more)
        def _():
            gather(r1 + 2, 0, _C0, b0, s_b0)

        wait(_C1, b1, s_b1)
        acc = reduce_rows(b1, _C1, acc)

        @pl.when(more)
        def _():
            gather(r1 + 2, _C0, _C1, b1, s_b1)

        store_row(r1, acc)
        return carry

    lax.fori_loop(0, nbb, iter_body, 0)

    pltpu.sync_copy(out_v, out_hbm.at[pl.ds(rbase, _RPT)])


_sc_gather_sum = pl.kernel(
    _sc_body,
    out_type=jax.ShapeDtypeStruct((_B, _D), jnp.float32),
    mesh=plsc.VectorSubcoreMesh(core_axis_name="c", subcore_axis_name="s",
                                num_cores=_NC, num_subcores=_NS),
    scratch_types=[
        pltpu.VMEM((_RPT, _L), jnp.int32),
        pltpu.VMEM((_C0, _D), jnp.float32),
        pltpu.VMEM((_C1, _D), jnp.float32),
        pltpu.VMEM((_C0, _D), jnp.float32),
        pltpu.VMEM((_C1, _D), jnp.float32),
        pltpu.VMEM((_RPT, _D), jnp.float32),
        pltpu.SemaphoreType.DMA,
        pltpu.SemaphoreType.DMA,
        pltpu.SemaphoreType.DMA,
        pltpu.SemaphoreType.DMA,
    ],
    compiler_params=pltpu.CompilerParams(use_tc_tiling_on_sc=False),
)


def _mlp_body(h_ref, w1_ref, b1_ref, w2_ref, b2_ref, o_ref):
    h = h_ref[...] * (1.0 / _L)
    z = jnp.dot(h, w1_ref[...], preferred_element_type=jnp.float32)
    z = jnp.maximum(z + b1_ref[...], 0.0)
    o_ref[...] = jnp.dot(z, w2_ref[...],
                         preferred_element_type=jnp.float32) + b2_ref[...]


_mlp_call = pl.pallas_call(
    _mlp_body,
    out_shape=jax.ShapeDtypeStruct((_B, _H), jnp.float32),
)


def kernel(x, emb_table, W1, b1, W2, b2):
    sums = _sc_gather_sum(x, emb_table)
    w2p = jnp.zeros((_H, _H), jnp.float32).at[:, :2].set(W2)
    b2p = jnp.zeros((1, _H), jnp.float32).at[0, :2].set(b2)
    out = _mlp_call(sums, W1, b1.reshape(1, _H), w2p, b2p)
    return out[:, :2]
